# baseline (speedup 1.0000x reference)
import functools

import jax
import jax.numpy as jnp
from jax import lax
from jax.experimental import pallas as pl
from jax.experimental.pallas import tpu as pltpu
from jax.experimental.pallas import tpu_sc as plsc

PIXEL_SCALE = 0.2619
N_VERTS = 32768
N = 2 * N_VERTS
NROWS = 2 * N
TOTAL_NNZ = 2 * N * 16

NC = 2
NS = 16
L = 16
NW = NC * NS
CHUNK = TOTAL_NNZ // NW
BLK = 8192
NBLK = CHUNK // BLK
ACC_SLICE = NROWS // NS


def _sc_partial(x_flat, vals, rows, cols):
    mesh = plsc.VectorSubcoreMesh(core_axis_name="c", subcore_axis_name="s")

    @functools.partial(
        pl.kernel,
        out_type=jax.ShapeDtypeStruct((NC, NROWS), jnp.float32),
        mesh=mesh,
        scratch_types=[
            pltpu.VMEM((N,), jnp.float32),
            pltpu.VMEM((BLK,), jnp.int32),
            pltpu.VMEM((BLK,), jnp.float32),
            pltpu.VMEM((BLK,), jnp.int32),
            pltpu.VMEM((BLK,), jnp.int32),
            pltpu.VMEM((BLK,), jnp.float32),
            pltpu.VMEM_SHARED((NROWS,), jnp.float32),
        ],
        compiler_params=pltpu.CompilerParams(needs_layout_passes=False),
    )
    def k(x_hbm, vals_hbm, rows_hbm, cols_hbm, out_hbm,
          x_v, cols_v, vals_v, rows_v, perm_v, prod_v, acc_sh):
        c = lax.axis_index("c")
        s = lax.axis_index("s")
        wid = c * NS + s
        base = wid * CHUNK

        pltpu.sync_copy(x_hbm, x_v)

        def zero_body(i, carry):
            prod_v[pl.ds(i * L, L)] = jnp.zeros((L,), jnp.float32)
            return carry
        lax.fori_loop(0, BLK // L, zero_body, 0)
        pltpu.sync_copy(prod_v, acc_sh.at[pl.ds(s * ACC_SLICE, ACC_SLICE)])
        plsc.subcore_barrier()

        def blk_body(b, carry):
            off = base + b * BLK
            pltpu.sync_copy(cols_hbm.at[pl.ds(off, BLK)], cols_v)
            pltpu.sync_copy(vals_hbm.at[pl.ds(off, BLK)], vals_v)
            pltpu.sync_copy(rows_hbm.at[pl.ds(off, BLK)], rows_v)

            def prod_body(i, inner):
                sl = pl.ds(i * L, L)
                r = rows_v[sl]
                j = r >> 16
                ii = (r >> 15) & 1
                v = r & 32767
                perm_v[sl] = (ii << 16) | (v << 1) | j
                xg = plsc.load_gather(x_v, [cols_v[sl]])
                prod_v[sl] = vals_v[sl] * xg
                return inner
            lax.fori_loop(0, BLK // L, prod_body, 0)

            pltpu.sync_copy(prod_v, acc_sh.at[perm_v], add=True)
            return carry
        lax.fori_loop(0, NBLK, blk_body, 0)

        plsc.subcore_barrier()
        pltpu.sync_copy(acc_sh.at[pl.ds(s * ACC_SLICE, ACC_SLICE)], prod_v)
        pltpu.sync_copy(prod_v, out_hbm.at[c, pl.ds(s * ACC_SLICE, ACC_SLICE)])

    return k(x_flat, vals, rows, cols)


def _combine(partial):
    p = partial.reshape(NC, NROWS // 128, 128)

    def body(p_ref, o_ref):
        o_ref[...] = (p_ref[0] + p_ref[1]) * (1.0 / PIXEL_SCALE)

    return pl.pallas_call(
        body,
        out_shape=jax.ShapeDtypeStruct((NROWS // 128, 128), jnp.float32),
    )(p)


def kernel(x, vals, rows, cols):
    x_flat = x.reshape(-1)
    partial = _sc_partial(x_flat, vals, rows, cols)
    return _combine(partial).reshape(2, N_VERTS, 2)


# speedup vs baseline: 1.0874x; 1.0874x over previous
import functools

import jax
import jax.numpy as jnp
from jax import lax
from jax.experimental import pallas as pl
from jax.experimental.pallas import tpu as pltpu
from jax.experimental.pallas import tpu_sc as plsc

PIXEL_SCALE = 0.2619
N_VERTS = 32768
N = 2 * N_VERTS
NROWS = 2 * N
TOTAL_NNZ = 2 * N * 16

NC = 2
NS = 16
L = 16
NW = NC * NS
CHUNK = TOTAL_NNZ // NW
BLK = 8192
NBLK = CHUNK // BLK
ACC_SLICE = NROWS // NS


def _sc_partial(x_flat, vals, rows, cols):
    mesh = plsc.VectorSubcoreMesh(core_axis_name="c", subcore_axis_name="s")

    @functools.partial(
        pl.kernel,
        out_type=jax.ShapeDtypeStruct((NC, NROWS), jnp.float32),
        mesh=mesh,
        scratch_types=[
            pltpu.VMEM((N,), jnp.float32),
            pltpu.VMEM((BLK,), jnp.int32),
            pltpu.VMEM((BLK,), jnp.float32),
            pltpu.VMEM((BLK,), jnp.int32),
            pltpu.VMEM((BLK,), jnp.int32),
            pltpu.VMEM((BLK,), jnp.float32),
            pltpu.VMEM_SHARED((NROWS,), jnp.float32),
        ],
        compiler_params=pltpu.CompilerParams(needs_layout_passes=False),
    )
    def k(x_hbm, vals_hbm, rows_hbm, cols_hbm, out_hbm,
          x_v, cols_v, vals_v, rows_v, perm_v, prod_v, acc_sh):
        c = lax.axis_index("c")
        s = lax.axis_index("s")
        wid = c * NS + s
        base = wid * CHUNK

        pltpu.sync_copy(x_hbm, x_v)

        def zero_body(i, carry):
            prod_v[pl.ds(i * L, L)] = jnp.zeros((L,), jnp.float32)
            return carry
        lax.fori_loop(0, BLK // L, zero_body, 0)
        pltpu.sync_copy(prod_v, acc_sh.at[pl.ds(s * ACC_SLICE, ACC_SLICE)])
        plsc.subcore_barrier()

        def blk_body(b, carry):
            off = base + b * BLK
            pltpu.sync_copy(cols_hbm.at[pl.ds(off, BLK)], cols_v)
            pltpu.sync_copy(vals_hbm.at[pl.ds(off, BLK)], vals_v)
            pltpu.sync_copy(rows_hbm.at[pl.ds(off, BLK)], rows_v)

            @plsc.parallel_loop(0, BLK, step=L, unroll=8)
            def prod_body(i):
                sl = pl.ds(i, L)
                r = rows_v[sl]
                j = r >> 16
                ii = (r >> 15) & 1
                v = r & 32767
                perm_v[sl] = (ii << 16) | (v << 1) | j
                xg = plsc.load_gather(x_v, [cols_v[sl]])
                prod_v[sl] = vals_v[sl] * xg

            pltpu.sync_copy(prod_v, acc_sh.at[perm_v], add=True)
            return carry
        lax.fori_loop(0, NBLK, blk_body, 0)

        plsc.subcore_barrier()
        pltpu.sync_copy(acc_sh.at[pl.ds(s * ACC_SLICE, ACC_SLICE)], prod_v)
        pltpu.sync_copy(prod_v, out_hbm.at[c, pl.ds(s * ACC_SLICE, ACC_SLICE)])

    return k(x_flat, vals, rows, cols)


def _combine(partial):
    p = partial.reshape(NC, NROWS // 128, 128)

    def body(p_ref, o_ref):
        o_ref[...] = (p_ref[0] + p_ref[1]) * (1.0 / PIXEL_SCALE)

    return pl.pallas_call(
        body,
        out_shape=jax.ShapeDtypeStruct((NROWS // 128, 128), jnp.float32),
    )(p)


def kernel(x, vals, rows, cols):
    x_flat = x.reshape(-1)
    partial = _sc_partial(x_flat, vals, rows, cols)
    return _combine(partial).reshape(2, N_VERTS, 2)


# speedup vs baseline: 1.7574x; 1.6161x over previous
import functools

import jax
import jax.numpy as jnp
from jax import lax
from jax.experimental import pallas as pl
from jax.experimental.pallas import tpu as pltpu
from jax.experimental.pallas import tpu_sc as plsc

PIXEL_SCALE = 0.2619
N_VERTS = 32768
N = 2 * N_VERTS
NROWS = 2 * N
TOTAL_NNZ = 2 * N * 16

NC = 2
NS = 16
L = 16
NW = NC * NS
CHUNK = TOTAL_NNZ // NW
BLK = 8192
NBLK = CHUNK // BLK
ACC_SLICE = NROWS // NS


def _sc_partial(x_flat, vals, rows, cols):
    mesh = plsc.VectorSubcoreMesh(core_axis_name="c", subcore_axis_name="s")

    @functools.partial(
        pl.kernel,
        out_type=jax.ShapeDtypeStruct((NC, NROWS), jnp.float32),
        mesh=mesh,
        scratch_types=[
            pltpu.VMEM((N,), jnp.float32),
            pltpu.VMEM((BLK,), jnp.int32),
            pltpu.VMEM((BLK,), jnp.float32),
            pltpu.VMEM((BLK,), jnp.int32),
            pltpu.VMEM((BLK,), jnp.int32),
            pltpu.VMEM((BLK,), jnp.float32),
            pltpu.VMEM_SHARED((NROWS,), jnp.float32),
        ],
        compiler_params=pltpu.CompilerParams(needs_layout_passes=False),
    )
    def k(x_hbm, vals_hbm, rows_hbm, cols_hbm, out_hbm,
          x_v, cols_v, vals_v, rows_v, perm_v, prod_v, acc_sh):
        c = lax.axis_index("c")
        s = lax.axis_index("s")
        wid = c * NS + s
        base = wid * CHUNK

        pltpu.sync_copy(x_hbm, x_v)

        def zero_body(i, carry):
            prod_v[pl.ds(i * L, L)] = jnp.zeros((L,), jnp.float32)
            return carry
        lax.fori_loop(0, BLK // L, zero_body, 0)
        pltpu.sync_copy(prod_v, acc_sh.at[pl.ds(s * ACC_SLICE, ACC_SLICE)])
        plsc.subcore_barrier()

        def blk_body(b, carry):
            off = base + b * BLK
            pltpu.sync_copy(cols_hbm.at[pl.ds(off, BLK)], cols_v)
            pltpu.sync_copy(vals_hbm.at[pl.ds(off, BLK)], vals_v)
            pltpu.sync_copy(rows_hbm.at[pl.ds(off, BLK)], rows_v)

            @plsc.parallel_loop(0, BLK, step=L, unroll=8)
            def prod_body(i):
                sl = pl.ds(i, L)
                xg = plsc.load_gather(x_v, [cols_v[sl]])
                prod_v[sl] = vals_v[sl] * xg

            pltpu.sync_copy(prod_v, acc_sh.at[rows_v], add=True)
            return carry
        lax.fori_loop(0, NBLK, blk_body, 0)

        plsc.subcore_barrier()
        pltpu.sync_copy(acc_sh.at[pl.ds(s * ACC_SLICE, ACC_SLICE)], prod_v)
        pltpu.sync_copy(prod_v, out_hbm.at[c, pl.ds(s * ACC_SLICE, ACC_SLICE)])

    return k(x_flat, vals, rows, cols)


def _combine(partial):
    p = partial.reshape(NC, NROWS // 128, 128)

    def body(p_ref, o_ref):
        o_ref[...] = (p_ref[0] + p_ref[1]) * (1.0 / PIXEL_SCALE)

    return pl.pallas_call(
        body,
        out_shape=jax.ShapeDtypeStruct((NROWS // 128, 128), jnp.float32),
    )(p)


def kernel(x, vals, rows, cols):
    x_flat = x.reshape(-1)
    partial = _sc_partial(x_flat, vals, rows, cols)
    grad = _combine(partial).reshape(2, 2, N_VERTS)
    return jnp.moveaxis(grad, 0, -1)


# speedup vs baseline: 2.2464x; 1.2783x over previous
import functools

import jax
import jax.numpy as jnp
from jax import lax
from jax.experimental import pallas as pl
from jax.experimental.pallas import tpu as pltpu
from jax.experimental.pallas import tpu_sc as plsc

PIXEL_SCALE = 0.2619
N_VERTS = 32768
N = 2 * N_VERTS
NROWS = 2 * N
TOTAL_NNZ = 2 * N * 16

NC = 2
NS = 16
L = 16
NW = NC * NS
CHUNK = TOTAL_NNZ // NW
BLK = 8192
NBLK = CHUNK // BLK
ACC_SLICE = NROWS // NS


def _sc_partial(x_flat, vals, rows, cols):
    mesh = plsc.VectorSubcoreMesh(core_axis_name="c", subcore_axis_name="s")

    @functools.partial(
        pl.kernel,
        out_type=jax.ShapeDtypeStruct((NC, NROWS), jnp.float32),
        mesh=mesh,
        scratch_types=[
            pltpu.VMEM((N,), jnp.float32),
            [pltpu.VMEM((BLK,), jnp.int32)] * 2,
            [pltpu.VMEM((BLK,), jnp.float32)] * 2,
            [pltpu.VMEM((BLK,), jnp.int32)] * 2,
            pltpu.VMEM((BLK,), jnp.float32),
            pltpu.VMEM_SHARED((NROWS,), jnp.float32),
            [pltpu.SemaphoreType.DMA] * 2,
        ],
        compiler_params=pltpu.CompilerParams(needs_layout_passes=False),
    )
    def k(x_hbm, vals_hbm, rows_hbm, cols_hbm, out_hbm,
          x_v, cols_v, vals_v, rows_v, prod_v, acc_sh, in_sem):
        c = lax.axis_index("c")
        s = lax.axis_index("s")
        wid = c * NS + s
        base = wid * CHUNK

        pltpu.sync_copy(x_hbm, x_v)

        def zero_body(i, carry):
            prod_v[pl.ds(i * L, L)] = jnp.zeros((L,), jnp.float32)
            return carry
        lax.fori_loop(0, BLK // L, zero_body, 0)
        pltpu.sync_copy(prod_v, acc_sh.at[pl.ds(s * ACC_SLICE, ACC_SLICE)])
        plsc.subcore_barrier()

        def issue_inputs(b):
            p = b % 2
            off = base + b * BLK
            return (
                pltpu.async_copy(cols_hbm.at[pl.ds(off, BLK)], cols_v[p], in_sem[p]),
                pltpu.async_copy(vals_hbm.at[pl.ds(off, BLK)], vals_v[p], in_sem[p]),
                pltpu.async_copy(rows_hbm.at[pl.ds(off, BLK)], rows_v[p], in_sem[p]),
            )

        in_desc = {0: issue_inputs(0)}
        for b in range(NBLK):
            p = b % 2
            if b + 1 < NBLK:
                in_desc[b + 1] = issue_inputs(b + 1)
            for d in in_desc.pop(b):
                d.wait()

            @plsc.parallel_loop(0, BLK, step=L, unroll=8)
            def prod_body(i, p=p):
                sl = pl.ds(i, L)
                xg = plsc.load_gather(x_v, [cols_v[p][sl]])
                prod_v[sl] = vals_v[p][sl] * xg

            pltpu.sync_copy(prod_v, acc_sh.at[rows_v[p]], add=True)

        plsc.subcore_barrier()
        pltpu.sync_copy(acc_sh.at[pl.ds(s * ACC_SLICE, ACC_SLICE)], prod_v)
        pltpu.sync_copy(prod_v, out_hbm.at[c, pl.ds(s * ACC_SLICE, ACC_SLICE)])

    return k(x_flat, vals, rows, cols)


def _combine(partial):
    p = partial.reshape(NC, NROWS // 128, 128)

    def body(p_ref, o_ref):
        o_ref[...] = (p_ref[0] + p_ref[1]) * (1.0 / PIXEL_SCALE)

    return pl.pallas_call(
        body,
        out_shape=jax.ShapeDtypeStruct((NROWS // 128, 128), jnp.float32),
    )(p)


def kernel(x, vals, rows, cols):
    x_flat = x.reshape(-1)
    partial = _sc_partial(x_flat, vals, rows, cols)
    grad = _combine(partial).reshape(2, 2, N_VERTS)
    return jnp.moveaxis(grad, 0, -1)


# speedup vs baseline: 2.5388x; 1.1302x over previous
import functools

import jax
import jax.numpy as jnp
from jax import lax
from jax.experimental import pallas as pl
from jax.experimental.pallas import tpu as pltpu
from jax.experimental.pallas import tpu_sc as plsc

PIXEL_SCALE = 0.2619
N_VERTS = 32768
N = 2 * N_VERTS
NROWS = 2 * N
TOTAL_NNZ = 2 * N * 16

NC = 2
NS = 16
L = 16
NW = NC * NS
CHUNK = TOTAL_NNZ // NW
BLK = 4096
NBLK = CHUNK // BLK
ACC_SLICE = NROWS // NS


def _sc_partial(x_flat, vals, rows, cols):
    mesh = plsc.VectorSubcoreMesh(core_axis_name="c", subcore_axis_name="s")

    @functools.partial(
        pl.kernel,
        out_type=jax.ShapeDtypeStruct((NC, NROWS), jnp.float32),
        mesh=mesh,
        scratch_types=[
            pltpu.VMEM((N,), jnp.float32),
            [pltpu.VMEM((BLK,), jnp.int32)] * 2,
            [pltpu.VMEM((BLK,), jnp.float32)] * 2,
            [pltpu.VMEM((BLK,), jnp.int32)] * 3,
            [pltpu.VMEM((BLK,), jnp.float32)] * 3,
            pltpu.VMEM_SHARED((NROWS,), jnp.float32),
            [pltpu.SemaphoreType.DMA] * 2,
            [pltpu.SemaphoreType.DMA] * 3,
        ],
        compiler_params=pltpu.CompilerParams(needs_layout_passes=False),
    )
    def k(x_hbm, vals_hbm, rows_hbm, cols_hbm, out_hbm,
          x_v, cols_v, vals_v, rows_v, prod_v, acc_sh, in_sem, sc_sem):
        c = lax.axis_index("c")
        s = lax.axis_index("s")
        wid = c * NS + s
        base = wid * CHUNK

        pltpu.sync_copy(x_hbm, x_v)

        def zero_body(i, carry):
            prod_v[0][pl.ds(i * L, L)] = jnp.zeros((L,), jnp.float32)
            return carry
        lax.fori_loop(0, BLK // L, zero_body, 0)
        for h in range(ACC_SLICE // BLK):
            pltpu.sync_copy(prod_v[0],
                            acc_sh.at[pl.ds(s * ACC_SLICE + h * BLK, BLK)])
        plsc.subcore_barrier()

        def issue_inputs(b):
            p2, p3 = b % 2, b % 3
            off = base + b * BLK
            return (
                pltpu.async_copy(cols_hbm.at[pl.ds(off, BLK)], cols_v[p2], in_sem[p2]),
                pltpu.async_copy(vals_hbm.at[pl.ds(off, BLK)], vals_v[p2], in_sem[p2]),
                pltpu.async_copy(rows_hbm.at[pl.ds(off, BLK)], rows_v[p3], in_sem[p2]),
            )

        in_desc = {0: issue_inputs(0), 1: issue_inputs(1)}
        sc_desc = {}
        for b in range(NBLK):
            p2, p3 = b % 2, b % 3
            if b + 1 < NBLK:
                if b >= 2:
                    sc_desc.pop(b - 2).wait()
                in_desc[b + 1] = issue_inputs(b + 1)
            for d in in_desc.pop(b):
                d.wait()

            @plsc.parallel_loop(0, BLK, step=L, unroll=8)
            def prod_body(i, p2=p2, p3=p3):
                sl = pl.ds(i, L)
                xg = plsc.load_gather(x_v, [cols_v[p2][sl]])
                prod_v[p3][sl] = vals_v[p2][sl] * xg

            sc_desc[b] = pltpu.async_copy(
                prod_v[p3], acc_sh.at[rows_v[p3]], sc_sem[p3], add=True)

        for b in sorted(sc_desc):
            sc_desc.pop(b).wait()

        plsc.subcore_barrier()
        for h in range(ACC_SLICE // BLK):
            off = s * ACC_SLICE + h * BLK
            pltpu.sync_copy(acc_sh.at[pl.ds(off, BLK)], prod_v[h % 3])
            pltpu.sync_copy(prod_v[h % 3], out_hbm.at[c, pl.ds(off, BLK)])

    return k(x_flat, vals, rows, cols)


def _combine(partial):
    p = partial.reshape(NC, NROWS // 128, 128)

    def body(p_ref, o_ref):
        o_ref[...] = (p_ref[0] + p_ref[1]) * (1.0 / PIXEL_SCALE)

    return pl.pallas_call(
        body,
        out_shape=jax.ShapeDtypeStruct((NROWS // 128, 128), jnp.float32),
    )(p)


def kernel(x, vals, rows, cols):
    x_flat = x.reshape(-1)
    partial = _sc_partial(x_flat, vals, rows, cols)
    grad = _combine(partial).reshape(2, 2, N_VERTS)
    return jnp.moveaxis(grad, 0, -1)


# speedup vs baseline: 2.5561x; 1.0068x over previous
import functools

import jax
import jax.numpy as jnp
from jax import lax
from jax.experimental import pallas as pl
from jax.experimental.pallas import tpu as pltpu
from jax.experimental.pallas import tpu_sc as plsc

PIXEL_SCALE = 0.2619
N_VERTS = 32768
N = 2 * N_VERTS
NROWS = 2 * N
TOTAL_NNZ = 2 * N * 16

NC = 2
NS = 16
L = 16
NW = NC * NS
CHUNK = TOTAL_NNZ // NW
BLK = 4096
NBLK = CHUNK // BLK
ACC_SLICE = NROWS // NS


def _sc_partial(x_flat, vals, rows, cols):
    mesh = plsc.VectorSubcoreMesh(core_axis_name="c", subcore_axis_name="s")

    @functools.partial(
        pl.kernel,
        out_type=jax.ShapeDtypeStruct((NC, NROWS), jnp.float32),
        mesh=mesh,
        scratch_types=[
            pltpu.VMEM((N,), jnp.float32),
            [pltpu.VMEM((BLK,), jnp.int32)] * 2,
            [pltpu.VMEM((BLK,), jnp.float32)] * 2,
            [pltpu.VMEM((BLK,), jnp.int32)] * 3,
            [pltpu.VMEM((BLK,), jnp.float32)] * 3,
            pltpu.VMEM_SHARED((NROWS,), jnp.float32),
            [pltpu.SemaphoreType.DMA] * 2,
            [pltpu.SemaphoreType.DMA] * 3,
        ],
        compiler_params=pltpu.CompilerParams(needs_layout_passes=False),
    )
    def k(x_hbm, vals_hbm, rows_hbm, cols_hbm, out_hbm,
          x_v, cols_v, vals_v, rows_v, prod_v, acc_sh, in_sem, sc_sem):
        c = lax.axis_index("c")
        s = lax.axis_index("s")
        wid = c * NS + s
        base = wid * CHUNK

        pltpu.sync_copy(x_hbm, x_v)

        def zero_body(i, carry):
            prod_v[0][pl.ds(i * L, L)] = jnp.zeros((L,), jnp.float32)
            return carry
        lax.fori_loop(0, BLK // L, zero_body, 0)
        for h in range(ACC_SLICE // BLK):
            pltpu.sync_copy(prod_v[0],
                            acc_sh.at[pl.ds(s * ACC_SLICE + h * BLK, BLK)])
        plsc.subcore_barrier()

        def issue_inputs(b):
            p2, p3 = b % 2, b % 3
            off = base + b * BLK
            return (
                pltpu.async_copy(cols_hbm.at[pl.ds(off, BLK)], cols_v[p2], in_sem[p2]),
                pltpu.async_copy(vals_hbm.at[pl.ds(off, BLK)], vals_v[p2], in_sem[p2]),
                pltpu.async_copy(rows_hbm.at[pl.ds(off, BLK)], rows_v[p3], in_sem[p2]),
            )

        in_desc = {0: issue_inputs(0), 1: issue_inputs(1)}
        sc_desc = {}
        for b in range(NBLK):
            p2, p3 = b % 2, b % 3
            if b + 1 < NBLK:
                in_desc[b + 1] = issue_inputs(b + 1)
            for d in in_desc.pop(b):
                d.wait()

            @plsc.parallel_loop(0, BLK, step=L, unroll=8)
            def prod_body(i, p2=p2, p3=p3):
                sl = pl.ds(i, L)
                xg = plsc.load_gather(x_v, [cols_v[p2][sl]])
                prod_v[p3][sl] = vals_v[p2][sl] * xg

            if b >= 1 and (b - 1) in sc_desc:
                sc_desc.pop(b - 1).wait()
            sc_desc[b] = pltpu.async_copy(
                prod_v[p3], acc_sh.at[rows_v[p3]], sc_sem[p3], add=True)

        for b in sorted(sc_desc):
            sc_desc.pop(b).wait()

        plsc.subcore_barrier()
        for h in range(ACC_SLICE // BLK):
            off = s * ACC_SLICE + h * BLK
            pltpu.sync_copy(acc_sh.at[pl.ds(off, BLK)], prod_v[h % 3])
            pltpu.sync_copy(prod_v[h % 3], out_hbm.at[c, pl.ds(off, BLK)])

    return k(x_flat, vals, rows, cols)


def _combine(partial):
    p = partial.reshape(NC, NROWS // 128, 128)

    def body(p_ref, o_ref):
        o_ref[...] = (p_ref[0] + p_ref[1]) * (1.0 / PIXEL_SCALE)

    return pl.pallas_call(
        body,
        out_shape=jax.ShapeDtypeStruct((NROWS // 128, 128), jnp.float32),
    )(p)


def kernel(x, vals, rows, cols):
    x_flat = x.reshape(-1)
    partial = _sc_partial(x_flat, vals, rows, cols)
    grad = _combine(partial).reshape(2, 2, N_VERTS)
    return jnp.moveaxis(grad, 0, -1)
